# Initial kernel scaffold; baseline (speedup 1.0000x reference)
import jax
import jax.numpy as jnp
from jax.experimental import pallas as pl


def kernel(userID, itemID, features, user_emb, item_emb, W1, b1, W2, b2):
    raise NotImplementedError("write your pallas kernel here")


# speedup vs baseline: 1.3419x; 1.3419x over previous
import functools

import jax
import jax.numpy as jnp
from jax import lax
from jax.experimental import pallas as pl
from jax.experimental.pallas import tpu as pltpu
from jax.experimental.pallas import tpu_sc as plsc

DIM = 32
CHUNK = 128


def _make_sc_gather(n_rows: int):
    info = plsc.get_sparse_core_info()
    nc, ns = info.num_cores, info.num_subcores
    nw = nc * ns
    n_chunk_rows = n_rows // CHUNK
    chunks_per_w = n_chunk_rows // nw
    assert chunks_per_w * nw == n_chunk_rows

    mesh = plsc.VectorSubcoreMesh(core_axis_name="c", subcore_axis_name="s")

    @functools.partial(
        pl.kernel,
        mesh=mesh,
        compiler_params=pltpu.CompilerParams(use_tc_tiling_on_sc=False),
        out_type=jax.ShapeDtypeStruct((n_chunk_rows, CHUNK, DIM), jnp.float32),
        scratch_types=[
            pltpu.VMEM((chunks_per_w, CHUNK), jnp.int32),
            pltpu.VMEM((chunks_per_w, CHUNK, DIM), jnp.float32),
            pltpu.SemaphoreType.DMA,
        ],
    )
    def gather_kernel(table_hbm, idx_hbm, out_hbm, idx_v, rows_v, sem):
        wid = lax.axis_index("s") * nc + lax.axis_index("c")
        base = wid * chunks_per_w
        pltpu.sync_copy(idx_hbm.at[pl.ds(base, chunks_per_w)], idx_v)
        copies = []
        for j in range(chunks_per_w):
            copies.append(
                pltpu.async_copy(table_hbm.at[idx_v.at[j]], rows_v.at[j], sem)
            )
        for c in copies:
            c.wait()
        pltpu.sync_copy(rows_v, out_hbm.at[pl.ds(base, chunks_per_w)])

    return gather_kernel


def _mlp_body(ue_ref, ie_ref, f_ref, w1u_ref, w1i_ref, w1f_ref, b1_ref,
              w2_ref, b2_ref, o_ref):
    h = (
        jnp.dot(ue_ref[...], w1u_ref[...], preferred_element_type=jnp.float32)
        + jnp.dot(ie_ref[...], w1i_ref[...], preferred_element_type=jnp.float32)
        + jnp.dot(f_ref[...], w1f_ref[...], preferred_element_type=jnp.float32)
        + b1_ref[...]
    )
    h = jnp.maximum(h, 0.0)
    o_ref[...] = (
        jnp.dot(h, w2_ref[...], preferred_element_type=jnp.float32)
        + b2_ref[...]
    )


def kernel(userID, itemID, features, user_emb, item_emb, W1, b1, W2, b2):
    del item_emb
    batch = userID.shape[0]
    n_rows = 2 * batch

    idx = jnp.concatenate([userID, itemID]).astype(jnp.int32)
    idx2d = idx.reshape(n_rows // CHUNK, CHUNK)

    gathered = _make_sc_gather(n_rows)(user_emb, idx2d)
    g = gathered.reshape(n_rows, DIM)

    blk = 2048
    nblk = batch // blk
    feat_dim = features.shape[1]
    hid = W1.shape[1]

    w1u = W1[:DIM]
    w1i = W1[DIM:2 * DIM]
    w1f = W1[2 * DIM:]
    b1r = b1.reshape(1, hid)
    b2r = b2.reshape(1, 1)

    out = pl.pallas_call(
        _mlp_body,
        grid=(nblk,),
        in_specs=[
            pl.BlockSpec((blk, DIM), lambda i: (i, 0)),
            pl.BlockSpec((blk, DIM), lambda i: (i + nblk, 0)),
            pl.BlockSpec((blk, feat_dim), lambda i: (i, 0)),
            pl.BlockSpec((DIM, hid), lambda i: (0, 0)),
            pl.BlockSpec((DIM, hid), lambda i: (0, 0)),
            pl.BlockSpec((feat_dim, hid), lambda i: (0, 0)),
            pl.BlockSpec((1, hid), lambda i: (0, 0)),
            pl.BlockSpec((hid, 1), lambda i: (0, 0)),
            pl.BlockSpec((1, 1), lambda i: (0, 0)),
        ],
        out_specs=pl.BlockSpec((blk, 1), lambda i: (i, 0)),
        out_shape=jax.ShapeDtypeStruct((batch, 1), jnp.float32),
    )(g, g, features, w1u, w1i, w1f, b1r, W2, b2r)

    return out
